# baseline (speedup 1.0000x reference)
import functools
import math

import jax
import jax.numpy as jnp
from jax import lax
from jax.experimental import pallas as pl
from jax.experimental.pallas import tpu as pltpu
from jax.experimental.pallas import tpu_sc as plsc

B = 1024
L = 200
D = 64
N_POS = 200
VOCAB = 1000000

_NC = 2
_NS = 16
_LANES = 16
_NW = _NC * _NS
_BB = B // 128
_L8 = L // 8
_UNITS = _L8 * _BB
_UPW = -(-_UNITS // _NW)
_SCALE = math.sqrt(float(D))


_C = 8192
_CH = _C // 2
_NBLK = -(-VOCAB // _C)
_ROWS2 = _NBLK * _CH


def _pack_body(a_ref, out_ref):
    at = a_ref[...].T
    out_ref[...] = jnp.concatenate([at[:_CH, :], at[_CH:, :]], axis=1)


def _pack(table_t):
    return pl.pallas_call(
        _pack_body,
        grid=(_NBLK,),
        in_specs=[pl.BlockSpec((D, _C), lambda j: (0, j))],
        out_specs=pl.BlockSpec((_CH, 128), lambda j: (j, 0)),
        out_shape=jax.ShapeDtypeStruct((_ROWS2, 128), jnp.float32),
    )(table_t)


def _pos_body(out_ref):
    pos = lax.broadcasted_iota(jnp.int32, (L, 128), 0).astype(jnp.float32)
    d = lax.broadcasted_iota(jnp.int32, (L, 128), 1)
    f = (d // 2).astype(jnp.float32)
    freq = jnp.exp(f * (-2.0 / D) * math.log(float(N_POS)))
    ang = pos * freq
    out_ref[...] = jnp.where(d % 2 == 0, jnp.sin(ang), jnp.cos(ang))


def _pos_enc():
    return pl.pallas_call(
        _pos_body,
        out_shape=jax.ShapeDtypeStruct((L, 128), jnp.float32),
    )()


def _sc_body(idx_t, pos, table2, out3, mask_out,
             pos_u, idx_u, row_u, mask_u, ubuf, g0, g1,
             gs0, gs1):
    gbufs = (g0, g1)
    gsems = (gs0, gs1)

    wid = lax.axis_index("s") * _NC + lax.axis_index("c")

    def unit_body(k, carry):
        u = wid + k * _NW

        @pl.when(u < _UNITS)
        def _():
            l8 = u // _BB
            bb = u % _BB
            l0 = l8 * 8
            b0 = bb * 128

            pltpu.sync_copy(idx_t.at[pl.ds(l0, 8), pl.ds(b0, 128)], idx_u)
            pltpu.sync_copy(pos.at[pl.ds(l0, 8)], pos_u)

            @plsc.parallel_loop(0, 8 * 8, unroll=4)
            def _(i):
                r = i // 8
                sl = pl.ds((i % 8) * _LANES, _LANES)
                v = idx_u[r, sl]
                row_u[r, sl] = (
                    lax.shift_left(lax.shift_right_logical(v, 13), 12)
                    + jnp.bitwise_and(v, 4095))
                mask_u[r, sl] = jnp.minimum(v, 1)

            pltpu.sync_copy(
                mask_u, mask_out.at[pl.ds(l0, 8), pl.ds(b0, 128)])

            def issue_gather(g, buf, sem):
                pltpu.async_copy(
                    table2.at[row_u.at[g % 8, pl.ds((g // 8) * 64, 64)]],
                    buf, sem)

            def wait_gather(buf, sem):
                pltpu.make_async_copy(
                    table2.at[pl.ds(0, 64)], buf, sem).wait()

            issue_gather(0, gbufs[0], gsems[0])

            def pair_body(m, carry2):
                for h in (0, 1):
                    g = 2 * m + h
                    r = lax.rem(g, 8)
                    boff = (g // 8) * 64

                    @pl.when(g + 1 < 16)
                    def _():
                        issue_gather(g + 1, gbufs[1 - h], gsems[1 - h])

                    wait_gather(gbufs[h], gsems[h])
                    gbuf = gbufs[h]
                    posr = [pos_u[r, pl.ds(c * _LANES, _LANES)]
                            for c in range(D // _LANES)]

                    @plsc.parallel_loop(0, 4)
                    def _(bg):
                        vec = idx_u[r, pl.ds(boff + bg * _LANES, _LANES)]
                        stv = jnp.bitwise_and(
                            lax.shift_right_logical(vec, 6), 64)
                        for j in range(_LANES):
                            st = stv[j]
                            b = bg * _LANES + j
                            for c in range(D // _LANES):
                                v = gbuf[b, pl.ds(st + c * _LANES, _LANES)]
                                ubuf[b, r, pl.ds(c * _LANES, _LANES)] = (
                                    v * _SCALE + posr[c])

                @pl.when(m == 3)
                def _():
                    pltpu.sync_copy(
                        ubuf, out3.at[pl.ds(b0, 64), pl.ds(l0, 8), :])

                @pl.when(m == 7)
                def _():
                    pltpu.sync_copy(
                        ubuf, out3.at[pl.ds(b0 + 64, 64), pl.ds(l0, 8), :])

                return carry2

            lax.fori_loop(0, 8, pair_body, 0)

        return carry

    lax.fori_loop(0, _UPW, unit_body, 0)


@jax.jit
def _run(idx_t, pos, table2):
    mesh = plsc.VectorSubcoreMesh(core_axis_name="c", subcore_axis_name="s")
    f = pl.kernel(
        _sc_body,
        out_type=(
            jax.ShapeDtypeStruct((B, L, D), jnp.float32),
            jax.ShapeDtypeStruct((L, B), jnp.int32),
        ),
        mesh=mesh,
        scratch_types=(
            pltpu.VMEM((8, 128), jnp.float32),
            pltpu.VMEM((8, 128), jnp.int32),
            pltpu.VMEM((8, 128), jnp.int32),
            pltpu.VMEM((8, 128), jnp.int32),
            pltpu.VMEM((64, 8, D), jnp.float32),
            pltpu.VMEM((64, 128), jnp.float32),
            pltpu.VMEM((64, 128), jnp.float32),
            pltpu.SemaphoreType.DMA,
            pltpu.SemaphoreType.DMA,
        ),
        compiler_params=pltpu.CompilerParams(
            use_tc_tiling_on_sc=True, needs_layout_passes=False),
    )
    return f(idx_t, pos, table2)


def kernel(indices, table):
    idx_t = indices.T.astype(jnp.int32)
    table2 = _pack(table.T)
    pos = _pos_enc()
    x, mask_t = _run(idx_t, pos, table2)
    mask = (mask_t != 0).T
    return (x, mask)


# speedup vs baseline: 1.1508x; 1.1508x over previous
import functools
import math

import jax
import jax.numpy as jnp
from jax import lax
from jax.experimental import pallas as pl
from jax.experimental.pallas import tpu as pltpu
from jax.experimental.pallas import tpu_sc as plsc

B = 1024
L = 200
D = 64
N_POS = 200
VOCAB = 1000000

_NC = 2
_NS = 16
_LANES = 16
_NW = _NC * _NS
_BB = B // 128
_L8 = L // 8
_UNITS = _L8 * _BB
_UPW = -(-_UNITS // _NW)
_SCALE = math.sqrt(float(D))


_C = 16384
_CH = _C // 2
_NBLK = -(-VOCAB // _C)
_ROWS2 = _NBLK * _CH


def _pack_body(a_ref, out_ref):
    at = a_ref[...].T
    out_ref[...] = jnp.concatenate([at[:_CH, :], at[_CH:, :]], axis=1)


def _pack(table_t):
    return pl.pallas_call(
        _pack_body,
        grid=(_NBLK,),
        in_specs=[pl.BlockSpec((D, _C), lambda j: (0, j))],
        out_specs=pl.BlockSpec((_CH, 128), lambda j: (j, 0)),
        out_shape=jax.ShapeDtypeStruct((_ROWS2, 128), jnp.float32),
    )(table_t)


def _pos_body(out_ref):
    pos = lax.broadcasted_iota(jnp.int32, (L, 128), 0).astype(jnp.float32)
    d = lax.broadcasted_iota(jnp.int32, (L, 128), 1)
    f = (d // 2).astype(jnp.float32)
    freq = jnp.exp(f * (-2.0 / D) * math.log(float(N_POS)))
    ang = pos * freq
    out_ref[...] = jnp.where(d % 2 == 0, jnp.sin(ang), jnp.cos(ang))


def _pos_enc():
    return pl.pallas_call(
        _pos_body,
        out_shape=jax.ShapeDtypeStruct((L, 128), jnp.float32),
    )()


def _sc_body(idx_t, pos, table2, out, mask_out,
             pos_v, idx_u, row_u, par_u, mask_u, g0, g1, g2, g3, w0, w1,
             gs0, gs1, gs2, gs3, ws0, ws1):
    gbufs = (g0, g1, g2, g3)
    wbufs = (w0, w1)
    gsems = (gs0, gs1, gs2, gs3)
    wsems = (ws0, ws1)

    wid = lax.axis_index("s") * _NC + lax.axis_index("c")

    pltpu.sync_copy(pos, pos_v)

    def unit_body(k, carry):
        u = wid + k * _NW

        @pl.when(u < _UNITS)
        def _():
            l8 = u // _BB
            bb = u % _BB
            l0 = l8 * 8
            b0 = bb * 128

            pltpu.sync_copy(idx_t.at[pl.ds(l0, 8), pl.ds(b0, 128)], idx_u)

            @plsc.parallel_loop(0, 8 * 8, unroll=4)
            def _(i):
                r = i // 8
                sl = pl.ds((i % 8) * _LANES, _LANES)
                v = idx_u[r, sl]
                row_u[r, sl] = (
                    lax.shift_left(lax.shift_right_logical(v, 14), 13)
                    + jnp.bitwise_and(v, 8191))
                par_u[r, sl] = jnp.bitwise_and(
                    lax.shift_right_logical(v, 7), 64)
                mask_u[r, sl] = jnp.minimum(v, 1)

            pltpu.sync_copy(
                mask_u, mask_out.at[pl.ds(l0, 8), pl.ds(b0, 128)])

            def issue_gather(r, buf, sem):
                pltpu.async_copy(table2.at[row_u.at[r]], buf, sem)

            def wait_gather(buf, sem):
                pltpu.make_async_copy(
                    table2.at[pl.ds(0, 128)], buf, sem).wait()

            def issue_write(l, buf, sem):
                pltpu.async_copy(
                    buf, out.at[l, :, pl.ds(b0, 128)], sem)

            def wait_write(buf, sem):
                pltpu.make_async_copy(
                    buf, out.at[0, :, pl.ds(0, 128)], sem).wait()

            issue_gather(0, gbufs[0], gsems[0])
            issue_gather(1, gbufs[1], gsems[1])
            for r in range(8):
                h4 = r % 4
                h = r % 2
                if r + 2 < 8:
                    issue_gather(r + 2, gbufs[(r + 2) % 4], gsems[(r + 2) % 4])
                wait_gather(gbufs[h4], gsems[h4])
                if r >= 2:
                    wait_write(wbufs[h], wsems[h])
                gbuf = gbufs[h4]
                wbuf = wbufs[h]
                par = [par_u[r, pl.ds(g * _LANES, _LANES)] for g in range(8)]
                toks = [
                    lax.iota(jnp.int32, _LANES) + g * _LANES
                    for g in range(8)
                ]
                lx = l0 + r
                lvec = jnp.broadcast_to(lx, (_LANES,))

                @plsc.parallel_loop(0, D, unroll=4)
                def _(d):
                    dvec = jnp.broadcast_to(d, (_LANES,))
                    p = plsc.load_gather(pos_v, [lvec, dvec])
                    for g in range(8):
                        val = plsc.load_gather(gbuf, [toks[g], par[g] + d])
                        wbuf[d, pl.ds(g * _LANES, _LANES)] = (
                            val * _SCALE + p)

                issue_write(lx, wbuf, wsems[h])
            for h in (0, 1):
                wait_write(wbufs[h], wsems[h])

        return carry

    lax.fori_loop(0, _UPW, unit_body, 0)


@jax.jit
def _run(idx_t, pos, table2):
    mesh = plsc.VectorSubcoreMesh(core_axis_name="c", subcore_axis_name="s")
    f = pl.kernel(
        _sc_body,
        out_type=(
            jax.ShapeDtypeStruct((L, D, B), jnp.float32),
            jax.ShapeDtypeStruct((L, B), jnp.int32),
        ),
        mesh=mesh,
        scratch_types=(
            pltpu.VMEM((L, 128), jnp.float32),
            pltpu.VMEM((8, 128), jnp.int32),
            pltpu.VMEM((8, 128), jnp.int32),
            pltpu.VMEM((8, 128), jnp.int32),
            pltpu.VMEM((8, 128), jnp.int32),
            pltpu.VMEM((128, 128), jnp.float32),
            pltpu.VMEM((128, 128), jnp.float32),
            pltpu.VMEM((128, 128), jnp.float32),
            pltpu.VMEM((128, 128), jnp.float32),
            pltpu.VMEM((D, 128), jnp.float32),
            pltpu.VMEM((D, 128), jnp.float32),
            pltpu.SemaphoreType.DMA,
            pltpu.SemaphoreType.DMA,
            pltpu.SemaphoreType.DMA,
            pltpu.SemaphoreType.DMA,
            pltpu.SemaphoreType.DMA,
            pltpu.SemaphoreType.DMA,
        ),
        compiler_params=pltpu.CompilerParams(
            use_tc_tiling_on_sc=True, needs_layout_passes=False),
    )
    return f(idx_t, pos, table2)


def kernel(indices, table):
    idx_t = indices.T.astype(jnp.int32)
    table2 = _pack(table.T)
    pos = _pos_enc()
    out_t, mask_t = _run(idx_t, pos, table2)
    x = out_t.transpose(2, 0, 1)
    mask = (mask_t != 0).T
    return (x, mask)


# speedup vs baseline: 1.1878x; 1.0321x over previous
import functools
import math

import jax
import jax.numpy as jnp
from jax import lax
from jax.experimental import pallas as pl
from jax.experimental.pallas import tpu as pltpu
from jax.experimental.pallas import tpu_sc as plsc

B = 1024
L = 200
D = 64
N_POS = 200
VOCAB = 1000000

_NC = 2
_NS = 16
_LANES = 16
_NW = _NC * _NS
_BB = B // 128
_L8 = L // 8
_UNITS = _L8 * _BB
_UPW = -(-_UNITS // _NW)
_SCALE = math.sqrt(float(D))


_C = 32768
_CH = _C // 2
_NBLK = -(-VOCAB // _C)
_ROWS2 = _NBLK * _CH


def _pack_body(a_ref, out_ref):
    at = a_ref[...].T
    out_ref[...] = jnp.concatenate([at[:_CH, :], at[_CH:, :]], axis=1)


def _pack(table_t):
    return pl.pallas_call(
        _pack_body,
        grid=(_NBLK,),
        in_specs=[pl.BlockSpec((D, _C), lambda j: (0, j))],
        out_specs=pl.BlockSpec((_CH, 128), lambda j: (j, 0)),
        out_shape=jax.ShapeDtypeStruct((_ROWS2, 128), jnp.float32),
    )(table_t)


def _pos_body(out_ref):
    pos = lax.broadcasted_iota(jnp.int32, (L, 128), 0).astype(jnp.float32)
    d = lax.broadcasted_iota(jnp.int32, (L, 128), 1)
    f = (d // 2).astype(jnp.float32)
    freq = jnp.exp(f * (-2.0 / D) * math.log(float(N_POS)))
    ang = pos * freq
    out_ref[...] = jnp.where(d % 2 == 0, jnp.sin(ang), jnp.cos(ang))


def _pos_enc():
    return pl.pallas_call(
        _pos_body,
        out_shape=jax.ShapeDtypeStruct((L, 128), jnp.float32),
    )()


def _sc_body(idx_t, pos, table2, out, mask_out,
             pos_v, idx_u, row_u, par_u, mask_u, g0, g1, g2, g3, w0, w1,
             gs0, gs1, gs2, gs3, ws0, ws1):
    gbufs = (g0, g1, g2, g3)
    wbufs = (w0, w1)
    gsems = (gs0, gs1, gs2, gs3)
    wsems = (ws0, ws1)

    wid = lax.axis_index("s") * _NC + lax.axis_index("c")

    pltpu.sync_copy(pos, pos_v)

    def unit_body(k, carry):
        u = wid + k * _NW

        @pl.when(u < _UNITS)
        def _():
            l8 = u // _BB
            bb = u % _BB
            l0 = l8 * 8
            b0 = bb * 128

            pltpu.sync_copy(idx_t.at[pl.ds(l0, 8), pl.ds(b0, 128)], idx_u)

            @plsc.parallel_loop(0, 8 * 8, unroll=4)
            def _(i):
                r = i // 8
                sl = pl.ds((i % 8) * _LANES, _LANES)
                v = idx_u[r, sl]
                row_u[r, sl] = (
                    lax.shift_left(lax.shift_right_logical(v, 15), 14)
                    + jnp.bitwise_and(v, 16383))
                par_u[r, sl] = jnp.bitwise_and(
                    lax.shift_right_logical(v, 8), 64)
                mask_u[r, sl] = jnp.minimum(v, 1)

            pltpu.sync_copy(
                mask_u, mask_out.at[pl.ds(l0, 8), pl.ds(b0, 128)])

            def issue_gather(r, buf, sem):
                pltpu.async_copy(table2.at[row_u.at[r]], buf, sem)

            def wait_gather(buf, sem):
                pltpu.make_async_copy(
                    table2.at[pl.ds(0, 128)], buf, sem).wait()

            def issue_write(l, buf, sem):
                pltpu.async_copy(
                    buf, out.at[l, :, pl.ds(b0, 128)], sem)

            def wait_write(buf, sem):
                pltpu.make_async_copy(
                    buf, out.at[0, :, pl.ds(0, 128)], sem).wait()

            issue_gather(0, gbufs[0], gsems[0])
            issue_gather(1, gbufs[1], gsems[1])
            for r in range(8):
                h4 = r % 4
                h = r % 2
                if r + 2 < 8:
                    issue_gather(r + 2, gbufs[(r + 2) % 4], gsems[(r + 2) % 4])
                wait_gather(gbufs[h4], gsems[h4])
                if r >= 2:
                    wait_write(wbufs[h], wsems[h])
                gbuf = gbufs[h4]
                wbuf = wbufs[h]
                par = [par_u[r, pl.ds(g * _LANES, _LANES)] for g in range(8)]
                toks = [
                    lax.iota(jnp.int32, _LANES) + g * _LANES
                    for g in range(8)
                ]
                lx = l0 + r
                lvec = jnp.broadcast_to(lx, (_LANES,))

                @plsc.parallel_loop(0, D, unroll=4)
                def _(d):
                    dvec = jnp.broadcast_to(d, (_LANES,))
                    p = plsc.load_gather(pos_v, [lvec, dvec])
                    for g in range(8):
                        val = plsc.load_gather(gbuf, [toks[g], par[g] + d])
                        wbuf[d, pl.ds(g * _LANES, _LANES)] = (
                            val * _SCALE + p)

                issue_write(lx, wbuf, wsems[h])
            for h in (0, 1):
                wait_write(wbufs[h], wsems[h])

        return carry

    lax.fori_loop(0, _UPW, unit_body, 0)


@jax.jit
def _run(idx_t, pos, table2):
    mesh = plsc.VectorSubcoreMesh(core_axis_name="c", subcore_axis_name="s")
    f = pl.kernel(
        _sc_body,
        out_type=(
            jax.ShapeDtypeStruct((L, D, B), jnp.float32),
            jax.ShapeDtypeStruct((L, B), jnp.int32),
        ),
        mesh=mesh,
        scratch_types=(
            pltpu.VMEM((L, 128), jnp.float32),
            pltpu.VMEM((8, 128), jnp.int32),
            pltpu.VMEM((8, 128), jnp.int32),
            pltpu.VMEM((8, 128), jnp.int32),
            pltpu.VMEM((8, 128), jnp.int32),
            pltpu.VMEM((128, 128), jnp.float32),
            pltpu.VMEM((128, 128), jnp.float32),
            pltpu.VMEM((128, 128), jnp.float32),
            pltpu.VMEM((128, 128), jnp.float32),
            pltpu.VMEM((D, 128), jnp.float32),
            pltpu.VMEM((D, 128), jnp.float32),
            pltpu.SemaphoreType.DMA,
            pltpu.SemaphoreType.DMA,
            pltpu.SemaphoreType.DMA,
            pltpu.SemaphoreType.DMA,
            pltpu.SemaphoreType.DMA,
            pltpu.SemaphoreType.DMA,
        ),
        compiler_params=pltpu.CompilerParams(
            use_tc_tiling_on_sc=True, needs_layout_passes=False),
    )
    return f(idx_t, pos, table2)


def kernel(indices, table):
    idx_t = indices.T.astype(jnp.int32)
    table2 = _pack(table.T)
    pos = _pos_enc()
    out_t, mask_t = _run(idx_t, pos, table2)
    x = out_t.transpose(2, 0, 1)
    mask = (mask_t != 0).T
    return (x, mask)
